# Initial kernel scaffold; baseline (speedup 1.0000x reference)
import jax
import jax.numpy as jnp
from jax.experimental import pallas as pl


def kernel(users, items, user_embedding, item_embedding, item_bias):
    raise NotImplementedError("write your pallas kernel here")


# speedup vs baseline: 1.4837x; 1.4837x over previous
import functools

import jax
import jax.numpy as jnp
from jax import lax
from jax.experimental import pallas as pl
from jax.experimental.pallas import tpu as pltpu
from jax.experimental.pallas import tpu_sc as plsc

K = 32
LANES = 16
NC = 2
NS = 16
NW = NC * NS
CHUNK = 512
ISLICE = 128
NSLICE = CHUNK // ISLICE


def _mf_body(users_hbm, items_hbm, ue_hbm, ie_hbm, ib_hbm, out_hbm,
             uidx_v, iidx_v, urows_v, irows_v, bias_v, out_v, sem):
    t_per_w = users_hbm.shape[0] // NW
    n_chunks = t_per_w // CHUNK
    wid = lax.axis_index("s") * NC + lax.axis_index("c")
    wbase = wid * t_per_w

    def chunk_body(c, _):
        base = pl.multiple_of(wbase + c * CHUNK, 8)

        for j in range(NSLICE):
            pltpu.sync_copy(users_hbm.at[pl.ds(base + j * ISLICE, ISLICE)],
                            uidx_v.at[j])
            pltpu.sync_copy(items_hbm.at[pl.ds(base + j * ISLICE, ISLICE)],
                            iidx_v.at[j])

        copies = []
        for j in range(NSLICE):
            sl = pl.ds(j * ISLICE, ISLICE)
            copies.append(pltpu.async_copy(ue_hbm.at[uidx_v.at[j]],
                                           urows_v.at[sl], sem))
            copies.append(pltpu.async_copy(ie_hbm.at[iidx_v.at[j]],
                                           irows_v.at[sl], sem))
            copies.append(pltpu.async_copy(ib_hbm.at[iidx_v.at[j]],
                                           bias_v.at[sl], sem))
        for cp in copies:
            cp.wait()

        lane_iota = lax.iota(jnp.int32, LANES)

        def group_body(g, _):
            rows = g * LANES + lane_iota
            acc = bias_v[pl.ds(g * LANES, LANES)]
            for k in range(K):
                kvec = jnp.full((LANES,), k, jnp.int32)
                u_c = plsc.load_gather(urows_v, [rows, kvec])
                i_c = plsc.load_gather(irows_v, [rows, kvec])
                acc = acc + u_c * i_c
            out_v[pl.ds(g * LANES, LANES)] = acc
            return 0

        lax.fori_loop(0, CHUNK // LANES, group_body, 0)
        pltpu.sync_copy(out_v, out_hbm.at[pl.ds(base, CHUNK)])
        return 0

    lax.fori_loop(0, n_chunks, chunk_body, 0)


def kernel(users, items, user_embedding, item_embedding, item_bias):
    shape = users.shape
    uflat = users.reshape(-1)
    iflat = items.reshape(-1)
    ibflat = item_bias.reshape(-1)
    total = uflat.shape[0]

    mesh = plsc.VectorSubcoreMesh(core_axis_name="c", subcore_axis_name="s",
                                  num_cores=NC, num_subcores=NS)
    run = pl.kernel(
        _mf_body,
        out_type=jax.ShapeDtypeStruct((total,), jnp.float32),
        mesh=mesh,
        compiler_params=pltpu.CompilerParams(needs_layout_passes=False,
                                             use_tc_tiling_on_sc=False),
        scratch_types=[
            pltpu.VMEM((NSLICE, ISLICE), jnp.int32),
            pltpu.VMEM((NSLICE, ISLICE), jnp.int32),
            pltpu.VMEM((CHUNK, K), jnp.float32),
            pltpu.VMEM((CHUNK, K), jnp.float32),
            pltpu.VMEM((CHUNK,), jnp.float32),
            pltpu.VMEM((CHUNK,), jnp.float32),
            pltpu.SemaphoreType.DMA,
        ],
    )
    out = run(uflat, iflat, user_embedding, item_embedding, ibflat)
    return out.reshape(shape)


# speedup vs baseline: 1.7413x; 1.1736x over previous
import jax
import jax.numpy as jnp
from jax import lax
from jax.experimental import pallas as pl
from jax.experimental.pallas import tpu as pltpu
from jax.experimental.pallas import tpu_sc as plsc

K = 32
LANES = 16
NC = 2
NS = 16
NW = NC * NS
CHUNK = 512
ISLICE = 128
NSLICE = CHUNK // ISLICE


def _mf_body(users_hbm, items_hbm, ue_hbm, ie_hbm, ib_hbm, out_hbm,
             uidx_all, iidx_all, urows0, irows0, bias0, urows1, irows1, bias1,
             out_v, sem0, sem1):
    rows_per_w = users_hbm.shape[0] // NW
    t_per_w = rows_per_w * ISLICE
    n_chunks = t_per_w // CHUNK
    n_pairs = n_chunks // 2
    wid = lax.axis_index("s") * NC + lax.axis_index("c")
    wbase = wid * t_per_w
    wrow = wid * rows_per_w

    pltpu.sync_copy(users_hbm.at[pl.ds(wrow, rows_per_w)], uidx_all)
    pltpu.sync_copy(items_hbm.at[pl.ds(wrow, rows_per_w)], iidx_all)

    def transfers(c, urows, irows, bias, sem):
        cps = []
        for j in range(NSLICE):
            row = c * NSLICE + j
            sl = pl.ds(j * ISLICE, ISLICE)
            cps.append(pltpu.make_async_copy(ue_hbm.at[uidx_all.at[row]],
                                             urows.at[sl], sem))
            cps.append(pltpu.make_async_copy(ie_hbm.at[iidx_all.at[row]],
                                             irows.at[sl], sem))
            cps.append(pltpu.make_async_copy(ib_hbm.at[iidx_all.at[row]],
                                             bias.at[sl], sem))
        return cps

    def fire(c, urows, irows, bias, sem):
        for cp in transfers(c, urows, irows, bias, sem):
            cp.start()

    def drain(c, urows, irows, bias, sem):
        for cp in transfers(c, urows, irows, bias, sem):
            cp.wait()

    lane_iota = lax.iota(jnp.int32, LANES)

    def compute(c, urows, irows, bias):
        def group_body(g, _):
            rows = g * LANES + lane_iota
            acc = bias[pl.ds(g * LANES, LANES)]
            for k in range(K):
                kvec = jnp.full((LANES,), k, jnp.int32)
                u_c = plsc.load_gather(urows, [rows, kvec])
                i_c = plsc.load_gather(irows, [rows, kvec])
                acc = acc + u_c * i_c
            out_v[pl.ds(g * LANES, LANES)] = acc
            return 0

        lax.fori_loop(0, CHUNK // LANES, group_body, 0)
        pltpu.sync_copy(out_v, out_hbm.at[pl.ds(wbase + c * CHUNK, CHUNK)])

    fire(0, urows0, irows0, bias0, sem0)

    def pair_body(p, _):
        c = p * 2
        fire(c + 1, urows1, irows1, bias1, sem1)
        drain(c, urows0, irows0, bias0, sem0)
        compute(c, urows0, irows0, bias0)

        @pl.when(p < n_pairs - 1)
        def _():
            fire(c + 2, urows0, irows0, bias0, sem0)

        drain(c + 1, urows1, irows1, bias1, sem1)
        compute(c + 1, urows1, irows1, bias1)
        return 0

    lax.fori_loop(0, n_pairs, pair_body, 0)


def kernel(users, items, user_embedding, item_embedding, item_bias):
    shape = users.shape
    uflat = users.reshape(-1, ISLICE)
    iflat = items.reshape(-1, ISLICE)
    ibflat = item_bias.reshape(-1)
    total = shape[0] * shape[1]
    rows_per_w = (total // ISLICE) // NW

    mesh = plsc.VectorSubcoreMesh(core_axis_name="c", subcore_axis_name="s",
                                  num_cores=NC, num_subcores=NS)
    run = pl.kernel(
        _mf_body,
        out_type=jax.ShapeDtypeStruct((total,), jnp.float32),
        mesh=mesh,
        compiler_params=pltpu.CompilerParams(needs_layout_passes=False,
                                             use_tc_tiling_on_sc=False),
        scratch_types=[
            pltpu.VMEM((rows_per_w, ISLICE), jnp.int32),
            pltpu.VMEM((rows_per_w, ISLICE), jnp.int32),
            pltpu.VMEM((CHUNK, K), jnp.float32),
            pltpu.VMEM((CHUNK, K), jnp.float32),
            pltpu.VMEM((CHUNK,), jnp.float32),
            pltpu.VMEM((CHUNK, K), jnp.float32),
            pltpu.VMEM((CHUNK, K), jnp.float32),
            pltpu.VMEM((CHUNK,), jnp.float32),
            pltpu.VMEM((CHUNK,), jnp.float32),
            pltpu.SemaphoreType.DMA,
            pltpu.SemaphoreType.DMA,
        ],
    )
    out = run(uflat, iflat, user_embedding, item_embedding, ibflat)
    return out.reshape(shape)
